# baseline (speedup 1.0000x reference)
import functools

import jax
import jax.numpy as jnp
from jax import lax
from jax.experimental import pallas as pl
from jax.experimental.pallas import tpu as pltpu
from jax.experimental.pallas import tpu_sc as plsc

N = 320000
D = 64
SEG = 10000
SEGP = 10240
NC = 2
NS = 16
NW = NC * NS
ROWS_W = N // NW
C = 128
K = ROWS_W // C
T = ROWS_W - K * C
SEG_T = SEGP // NS


def _sc_body(emb_hbm, idxa_hbm, idxb_hbm, out_hbm, idx_v, idx_t, acc, buf0,
             buf1, sem0, sem1):
    c = lax.axis_index("c")
    s = lax.axis_index("s")
    wid = c * NS + s
    row_base = wid * ROWS_W

    zeros = jnp.zeros((16,), jnp.float32)

    def _zrow(i, _):
        for j in range(D // 16):
            buf0[i, pl.ds(j * 16, 16)] = zeros
        return 0

    lax.fori_loop(0, C, _zrow, 0)
    for r in range(SEG_T // C):
        pltpu.sync_copy(buf0, acc.at[pl.ds(s * SEG_T + r * C, C)])

    pltpu.sync_copy(idxa_hbm.at[wid], idx_v)
    pltpu.sync_copy(idxb_hbm.at[wid], idx_t)

    plsc.subcore_barrier()

    def _gather(g, buf, sem):
        pltpu.async_copy(emb_hbm.at[pl.ds(row_base + g * C, C), 0], buf, sem)

    def _gwait(buf, sem):
        pltpu.make_async_copy(emb_hbm.at[pl.ds(row_base, C), 0], buf, sem).wait()

    def _scat(g, buf):
        pltpu.sync_copy(buf, acc.at[idx_v.at[g]], add=True)

    _gather(0, buf0, sem0)

    def _pair(i, _):
        g = 2 * i
        _gather(g + 1, buf1, sem1)
        _gwait(buf0, sem0)
        _scat(g, buf0)
        _gather(g + 2, buf0, sem0)
        _gwait(buf1, sem1)
        _scat(g + 1, buf1)
        return 0

    lax.fori_loop(0, (K - 2) // 2, _pair, 0)
    _gather(K - 1, buf1, sem1)
    _gwait(buf0, sem0)
    _scat(K - 2, buf0)
    pltpu.async_copy(emb_hbm.at[pl.ds(row_base + K * C, T), 0],
                     buf0.at[pl.ds(0, T)], sem0)
    _gwait(buf1, sem1)
    _scat(K - 1, buf1)
    pltpu.make_async_copy(emb_hbm.at[pl.ds(row_base, T), 0],
                          buf0.at[pl.ds(0, T)], sem0).wait()
    pltpu.sync_copy(buf0.at[pl.ds(0, T)], acc.at[idx_t.at[0]], add=True)

    plsc.subcore_barrier()

    pltpu.sync_copy(
        acc.at[pl.ds(s * SEG_T, SEG_T)],
        out_hbm.at[c, pl.ds(s * SEG_T, SEG_T)],
    )


@functools.partial(
    pl.kernel,
    mesh=plsc.VectorSubcoreMesh(core_axis_name="c", subcore_axis_name="s"),
    out_type=jax.ShapeDtypeStruct((NC, SEGP, D), jnp.float32),
    scratch_types=[
        pltpu.VMEM((K, C), jnp.int32),
        pltpu.VMEM((1, T), jnp.int32),
        pltpu.VMEM_SHARED((SEGP, D), jnp.float32),
        pltpu.VMEM((C, D), jnp.float32),
        pltpu.VMEM((C, D), jnp.float32),
        pltpu.SemaphoreType.DMA,
        pltpu.SemaphoreType.DMA,
    ],
)
def _sc_segment_sum(emb_hbm, idxa_hbm, idxb_hbm, out_hbm, idx_v, idx_t, acc,
                    buf0, buf1, sem0, sem1):
    _sc_body(emb_hbm, idxa_hbm, idxb_hbm, out_hbm, idx_v, idx_t, acc, buf0,
             buf1, sem0, sem1)


def _combine_body(a_ref, b_ref, o_ref):
    o_ref[...] = a_ref[0] + b_ref[0]


def kernel(embedding, fragment_regionxcell_ix, cell_n, region_n):
    del cell_n, region_n
    embedding = embedding.reshape(N, 2, 64)
    idx2 = fragment_regionxcell_ix.reshape(NW, ROWS_W)
    idxa = idx2[:, : K * C].reshape(NW, K, C)
    idxb = idx2[:, K * C :].reshape(NW, 1, T)
    partials = _sc_segment_sum(embedding, idxa, idxb)
    out = pl.pallas_call(
        _combine_body,
        grid=(10,),
        in_specs=[
            pl.BlockSpec((1, SEG // 10, D), lambda i: (0, i, 0)),
            pl.BlockSpec((1, SEG // 10, D), lambda i: (1, i, 0)),
        ],
        out_specs=pl.BlockSpec((SEG // 10, D), lambda i: (i, 0)),
        out_shape=jax.ShapeDtypeStruct((SEG, D), jnp.float32),
    )(partials, partials)
    return jnp.concatenate([out, out], axis=-1).reshape(10, 1000, 128)


# speedup vs baseline: 4.8037x; 4.8037x over previous
import functools

import jax
import jax.numpy as jnp
from jax import lax
from jax.experimental import pallas as pl
from jax.experimental.pallas import tpu as pltpu
from jax.experimental.pallas import tpu_sc as plsc

N = 320000
D = 128
SEG = 10000
SEGP = 10240
NC = 2
NS = 16
NW = NC * NS
ROWS_W = N // NW
C = 128
K = ROWS_W // C
T = ROWS_W - K * C
SEG_T = SEGP // NS


def _sc_body(emb_hbm, idxa_hbm, idxb_hbm, out_hbm, idx_v, idx_t, acc, buf0,
             buf1, sem0, sem1):
    c = lax.axis_index("c")
    s = lax.axis_index("s")
    wid = c * NS + s
    row_base = wid * ROWS_W

    zeros = jnp.zeros((16,), jnp.float32)

    def _zrow(i, _):
        for j in range(D // 16):
            buf0[i, pl.ds(j * 16, 16)] = zeros
        return 0

    pltpu.async_copy(emb_hbm.at[pl.ds(row_base, C)], buf1, sem1)
    lax.fori_loop(0, C, _zrow, 0)
    for r in range(SEG_T // C):
        pltpu.sync_copy(buf0, acc.at[pl.ds(s * SEG_T + r * C, C)])

    pltpu.sync_copy(idxa_hbm.at[wid], idx_v)
    pltpu.sync_copy(idxb_hbm.at[wid], idx_t)

    plsc.subcore_barrier()

    def _gather(g, buf, sem):
        pltpu.async_copy(emb_hbm.at[pl.ds(row_base + g * C, C)], buf, sem)

    def _gwait(buf, sem):
        pltpu.make_async_copy(emb_hbm.at[pl.ds(row_base, C)], buf, sem).wait()

    def _scat(g, buf):
        pltpu.sync_copy(buf, acc.at[idx_v.at[g]], add=True)

    def _pair(i, _):
        g = 2 * i
        _gather(g + 1, buf0, sem0)
        _gwait(buf1, sem1)
        _scat(g, buf1)
        _gather(g + 2, buf1, sem1)
        _gwait(buf0, sem0)
        _scat(g + 1, buf0)
        return 0

    lax.fori_loop(0, (K - 2) // 2, _pair, 0)
    _gather(K - 1, buf0, sem0)
    _gwait(buf1, sem1)
    _scat(K - 2, buf1)
    pltpu.async_copy(emb_hbm.at[pl.ds(row_base + K * C, T)],
                     buf1.at[pl.ds(0, T)], sem1)
    _gwait(buf0, sem0)
    _scat(K - 1, buf0)
    pltpu.make_async_copy(emb_hbm.at[pl.ds(row_base, T)],
                          buf1.at[pl.ds(0, T)], sem1).wait()
    pltpu.sync_copy(buf1.at[pl.ds(0, T)], acc.at[idx_t.at[0]], add=True)

    plsc.subcore_barrier()

    pltpu.sync_copy(
        acc.at[pl.ds(s * SEG_T, SEG_T)],
        out_hbm.at[c, pl.ds(s * SEG_T, SEG_T)],
    )


@functools.partial(
    pl.kernel,
    mesh=plsc.VectorSubcoreMesh(core_axis_name="c", subcore_axis_name="s"),
    out_type=jax.ShapeDtypeStruct((NC, SEGP, D), jnp.float32),
    scratch_types=[
        pltpu.VMEM((K, C), jnp.int32),
        pltpu.VMEM((1, T), jnp.int32),
        pltpu.VMEM_SHARED((SEGP, D), jnp.float32),
        pltpu.VMEM((C, D), jnp.float32),
        pltpu.VMEM((C, D), jnp.float32),
        pltpu.SemaphoreType.DMA,
        pltpu.SemaphoreType.DMA,
    ],
)
def _sc_segment_sum(emb_hbm, idxa_hbm, idxb_hbm, out_hbm, idx_v, idx_t, acc,
                    buf0, buf1, sem0, sem1):
    _sc_body(emb_hbm, idxa_hbm, idxb_hbm, out_hbm, idx_v, idx_t, acc, buf0,
             buf1, sem0, sem1)


def _combine_body(a_ref, b_ref, o_ref):
    o_ref[...] = a_ref[0] + b_ref[0]


def kernel(embedding, fragment_regionxcell_ix, cell_n, region_n):
    del cell_n, region_n
    idx2 = fragment_regionxcell_ix.reshape(NW, ROWS_W)
    idxa = idx2[:, : K * C].reshape(NW, K, C)
    idxb = idx2[:, K * C :].reshape(NW, 1, T)
    partials = _sc_segment_sum(embedding, idxa, idxb)
    out = pl.pallas_call(
        _combine_body,
        grid=(10,),
        in_specs=[
            pl.BlockSpec((1, SEG // 10, D), lambda i: (0, i, 0)),
            pl.BlockSpec((1, SEG // 10, D), lambda i: (1, i, 0)),
        ],
        out_specs=pl.BlockSpec((SEG // 10, D), lambda i: (i, 0)),
        out_shape=jax.ShapeDtypeStruct((SEG, D), jnp.float32),
    )(partials, partials)
    return out.reshape(10, 1000, D)
